# baseline (speedup 1.0000x reference)
import functools
import jax
import jax.numpy as jnp
from jax import lax
from jax.experimental import pallas as pl
from jax.experimental.pallas import tpu as pltpu
from jax.experimental.pallas import tpu_sc as plsc

N_EMBED = 100000
N_LAYER = 16
CH = 64
D = N_LAYER * CH
B_TOT = 16384
S = 4
LPS = N_LAYER // S
DS = LPS * CH
NC, NS, LANES = 2, 16, 16
NW = NC * NS
BPW = B_TOT // NW
CB = 32
NCHUNK = BPW // CB
WB = 128
NCHW = WB // CB
NWIN = BPW // WB
NBUF = 2

_mesh = plsc.VectorSubcoreMesh(core_axis_name="c", subcore_axis_name="s")


@functools.partial(
    pl.kernel,
    out_type=jax.ShapeDtypeStruct((DS, B_TOT), jnp.float32),
    mesh=_mesh,
    scratch_types=[
        pltpu.VMEM((BPW,), jnp.float32),
        [pltpu.VMEM((CB,), jnp.int32) for _ in range(NBUF)],
        [pltpu.VMEM((CB,), jnp.int32) for _ in range(NBUF)],
        [pltpu.VMEM((CB,), jnp.float32) for _ in range(NBUF)],
        [pltpu.VMEM((CB,), jnp.float32) for _ in range(NBUF)],
        [pltpu.VMEM((CB, DS), jnp.float32) for _ in range(NBUF)],
        [pltpu.VMEM((CB, DS), jnp.float32) for _ in range(NBUF)],
        [pltpu.VMEM((DS, WB), jnp.float32) for _ in range(NBUF)],
        [pltpu.SemaphoreType.DMA for _ in range(NBUF)],
        [pltpu.SemaphoreType.DMA for _ in range(NBUF)],
    ],
    compiler_params=pltpu.CompilerParams(needs_layout_passes=False),
)
def _embed_slice(ori_hbm, table_hbm, out_hbm,
                 ori_v, idxl, idxr, wl_v, wr_v, bufl, bufr, slab, gsem, osem):
    wid = lax.axis_index("s") * NC + lax.axis_index("c")
    base = wid * BPW
    pltpu.sync_copy(ori_hbm.at[pl.ds(base, BPW)], ori_v)

    def stage_indices(q, b):
        for g in range(CB // LANES):
            o = ori_v[pl.ds(q * CB + g * LANES, LANES)]
            s = (o + 1.0) * 0.5 * float(N_EMBED)
            i0 = s.astype(jnp.int32)
            f = s - i0.astype(jnp.float32)
            il = jnp.where(i0 >= N_EMBED, i0 - N_EMBED, i0)
            i1 = i0 + 1
            ir = jnp.where(i1 >= N_EMBED, i1 - N_EMBED, i1)
            sl = pl.ds(g * LANES, LANES)
            idxl[b][sl] = il
            idxr[b][sl] = ir
            wl_v[b][sl] = 1.0 - f
            wr_v[b][sl] = f

    def start_gathers(b):
        pltpu.async_copy(table_hbm.at[idxl[b]], bufl[b], gsem[b])
        pltpu.async_copy(table_hbm.at[idxr[b]], bufr[b], gsem[b])

    def wait_gathers(b):
        pltpu.make_async_copy(table_hbm.at[idxl[b]], bufl[b], gsem[b]).wait()
        pltpu.make_async_copy(table_hbm.at[idxr[b]], bufr[b], gsem[b]).wait()

    lane_iota = lax.iota(jnp.int32, LANES)

    def interp_chunk(b, sw, col0):
        def row_body(j, carry):
            jv = jnp.zeros((LANES,), jnp.int32) + j
            wl = plsc.load_gather(wl_v[b], [jv])
            wr = plsc.load_gather(wr_v[b], [jv])
            cv = jv + col0

            def col_body(v, carry2):
                sl = pl.ds(v * LANES, LANES)
                val = bufl[b][j, sl] * wl + bufr[b][j, sl] * wr
                rows = lane_iota + v * LANES
                plsc.store_scatter(slab[sw], [rows, cv], val)
                return carry2

            lax.fori_loop(0, DS // LANES, col_body, 0, unroll=8)
            return carry

        lax.fori_loop(0, CB, row_body, 0)

    stage_indices(0, 0)
    start_gathers(0)

    def outer(w0, carry):
        for sw in range(NBUF):
            w = w0 + sw

            @pl.when(w >= NBUF)
            def _():
                pltpu.make_async_copy(
                    slab[sw], out_hbm.at[:, pl.ds(base, WB)], osem[sw]).wait()

            for c in range(NCHW):
                q = w * NCHW + c
                b = c % NBUF

                @pl.when(q + 1 < NCHUNK)
                def _():
                    stage_indices(q + 1, 1 - b)
                    start_gathers(1 - b)

                wait_gathers(b)
                interp_chunk(b, sw, c * CB)

            pltpu.async_copy(
                slab[sw], out_hbm.at[:, pl.ds(base + w * WB, WB)], osem[sw])
        return carry

    lax.fori_loop(0, NWIN // NBUF, lambda i, c: outer(i * NBUF, c), 0)

    for sw in range(NBUF):
        pltpu.make_async_copy(
            slab[sw], out_hbm.at[:, pl.ds(base, WB)], osem[sw]).wait()


def kernel(ori, embeds):
    slabs = []
    for si in range(S):
        table_s = embeds[:, si * LPS:(si + 1) * LPS, :].reshape(N_EMBED, DS)
        slabs.append(_embed_slice(ori, table_s))
    out_t = jnp.concatenate(slabs, axis=0)
    return out_t.reshape(N_LAYER, CH, B_TOT).transpose(2, 0, 1)
